# Initial kernel scaffold; baseline (speedup 1.0000x reference)
import jax
import jax.numpy as jnp
from jax.experimental import pallas as pl


def kernel(x, edge_index, edge_attr, batch, data, params):
    raise NotImplementedError("write your pallas kernel here")


# speedup vs baseline: 5.4489x; 5.4489x over previous
import functools

import jax
import jax.numpy as jnp
from jax import lax
from jax.experimental import pallas as pl
from jax.experimental.pallas import tpu as pltpu
from jax.experimental.pallas import tpu_sc as plsc

N = 10000
E = 320000
X_DIM = 128
EDGE_DIM = 16
C = 64
N_GRAPHS = 128
EPS_GEN = 1e-7
BN_EPS = 1e-5

NC = 2
NS = 16
NW = NC * NS
K = 96
CPT = 106
EPAD = NW * K * CPT
DUMMY = N
ACC_ROWS = 10112
RPS = ACC_ROWS // NS


def _edge_pass_body(h_hbm, ea_hbm, src_hbm, dst_hbm, t_hbm, out_hbm,
                    acc, srcv, dstv, eav, hsv, outv, zv, tv, sem):
    c = lax.axis_index("c")
    s = lax.axis_index("s")
    wid = c * NS + s

    pltpu.sync_copy(t_hbm, tv)

    zero = jnp.zeros((16,), jnp.float32)

    def _zb(i, carry):
        r = i // 8
        j = i - r * 8
        zv[r, pl.ds(j * 16, 16)] = zero
        return carry

    lax.fori_loop(0, 8 * 8, _zb, 0)
    row0 = s * RPS

    def _zc(i, carry):
        pltpu.sync_copy(zv, acc.at[pl.ds(row0 + i * 8, 8), :])
        return carry

    lax.fori_loop(0, RPS // 8, _zc, 0)
    plsc.subcore_barrier()

    tvec = tv[...]

    def _chunk(t, carry):
        base = (wid * CPT + t) * K
        pltpu.sync_copy(src_hbm.at[pl.ds(base, K)], srcv)
        pltpu.sync_copy(dst_hbm.at[pl.ds(base, K)], dstv)
        pltpu.sync_copy(ea_hbm.at[pl.ds(base, K), :], eav)
        pltpu.async_copy(h_hbm.at[srcv], hsv, sem).wait()

        def _edge(k, carry2):
            for j in range(4):
                hs = hsv[k, pl.ds(j * 16, 16)]
                ev = eav[k, pl.ds(j * 16, 16)]
                m = jnp.maximum(hs + ev, 0.0) + EPS_GEN
                e = jnp.exp(m * tvec)
                outv[k, pl.ds(j * 16, 16)] = m * e
                outv[k, pl.ds(C + j * 16, 16)] = e
            return carry2

        lax.fori_loop(0, K, _edge, 0)
        pltpu.sync_copy(outv, acc.at[dstv], add=True)
        return carry

    lax.fori_loop(0, CPT, _chunk, 0)
    plsc.subcore_barrier()

    pltpu.sync_copy(acc.at[pl.ds(row0, RPS), :],
                    out_hbm.at[c, pl.ds(row0, RPS), :])


@functools.lru_cache(maxsize=1)
def _get_edge_pass():
  return pl.kernel(
    _edge_pass_body,
    out_type=jax.ShapeDtypeStruct((NC, ACC_ROWS, 2 * C), jnp.float32),
    mesh=plsc.VectorSubcoreMesh(
        core_axis_name="c", subcore_axis_name="s", num_cores=NC,
        num_subcores=NS),
    scratch_types=[
        pltpu.VMEM_SHARED((ACC_ROWS, 2 * C), jnp.float32),
        pltpu.VMEM((K,), jnp.int32),
        pltpu.VMEM((K,), jnp.int32),
        pltpu.VMEM((K, C), jnp.float32),
        pltpu.VMEM((K, 2 * C), jnp.float32),
        pltpu.VMEM((K, 2 * C), jnp.float32),
        pltpu.VMEM((8, 2 * C), jnp.float32),
        pltpu.VMEM((16,), jnp.float32),
        pltpu.SemaphoreType.DMA,
    ],
  )


def _node_embed_body(x_ref, w_ref, b_ref, o_ref):
    o_ref[:, :C] = (
        jnp.dot(x_ref[...], w_ref[...], preferred_element_type=jnp.float32)
        + b_ref[...]
    )
    o_ref[:, C:] = jnp.zeros((N, C), jnp.float32)


def _edge_embed_body(a_ref, w_ref, b_ref, o_ref):
    o_ref[...] = (
        jnp.dot(a_ref[...], w_ref[...], preferred_element_type=jnp.float32)
        + b_ref[...]
    )


def _layer_body(acc_ref, h_ref, wc1, bc1, gc, bc, wc2, bc2,
                wm1, bm1, gm, bm, wm2, bm2, o_ref):
    numer = acc_ref[0, :N, :C] + acc_ref[1, :N, :C]
    denom = acc_ref[0, :N, C:] + acc_ref[1, :N, C:]
    h = h_ref[:, :C]
    aggr = numer / (denom + 1e-16)
    out = h + aggr
    hh = jnp.dot(out, wc1[...], preferred_element_type=jnp.float32) + bc1[...]
    mu = jnp.mean(hh, axis=0, keepdims=True)
    var = jnp.mean((hh - mu) ** 2, axis=0, keepdims=True)
    hh = (hh - mu) / jnp.sqrt(var + BN_EPS) * gc[...] + bc[...]
    hh = jnp.maximum(hh, 0.0)
    h2 = jnp.dot(hh, wc2[...], preferred_element_type=jnp.float32) + bc2[...]
    hm = jnp.dot(h2, wm1[...], preferred_element_type=jnp.float32) + bm1[...]
    mu2 = jnp.mean(hm, axis=0, keepdims=True)
    var2 = jnp.mean((hm - mu2) ** 2, axis=0, keepdims=True)
    hm = (hm - mu2) / jnp.sqrt(var2 + BN_EPS) * gm[...] + bm[...]
    hm = jnp.where(hm >= 0, hm, 0.01 * hm)
    h2 = jnp.dot(hm, wm2[...], preferred_element_type=jnp.float32) + bm2[...]
    o_ref[:, :C] = h2 + h
    o_ref[:, C:] = jnp.zeros((N, C), jnp.float32)


def _pool_body(h_ref, batch_ref, w_ref, b_ref, o_ref):
    gids = lax.broadcasted_iota(jnp.int32, (N_GRAPHS, 1), 0)
    bm = (batch_ref[...] == gids).astype(jnp.float32)
    ssum = jnp.dot(bm, h_ref[:, :C], preferred_element_type=jnp.float32)
    cnt = jnp.sum(bm, axis=1, keepdims=True)
    pooled = ssum / jnp.maximum(cnt, 1.0)
    o_ref[...] = (
        jnp.dot(pooled, w_ref[...], preferred_element_type=jnp.float32)
        + b_ref[...]
    )


def kernel(x, edge_index, edge_attr, batch, data, params):
    src = jnp.pad(edge_index[0], (0, EPAD - E))
    dst = jnp.pad(edge_index[1], (0, EPAD - E), constant_values=DUMMY)
    ea_in = jnp.pad(edge_attr, ((0, EPAD - E), (0, 0)))

    h = pl.pallas_call(
        _node_embed_body,
        out_shape=jax.ShapeDtypeStruct((N, 2 * C), jnp.float32),
    )(x, params['W_node'], params['b_node'].reshape(1, C))

    eb = 1536
    ea = pl.pallas_call(
        _edge_embed_body,
        grid=(EPAD // eb,),
        in_specs=[
            pl.BlockSpec((eb, EDGE_DIM), lambda i: (i, 0)),
            pl.BlockSpec((EDGE_DIM, C), lambda i: (0, 0)),
            pl.BlockSpec((1, C), lambda i: (0, 0)),
        ],
        out_specs=pl.BlockSpec((eb, C), lambda i: (i, 0)),
        out_shape=jax.ShapeDtypeStruct((EPAD, C), jnp.float32),
    )(ea_in, params['W_edge'], params['b_edge'].reshape(1, C))

    for lp in params['layers']:
        t_arr = jnp.broadcast_to(lp['t'], (16,)).astype(jnp.float32)
        acc = _get_edge_pass()(h, ea, src, dst, t_arr)
        h = pl.pallas_call(
            _layer_body,
            out_shape=jax.ShapeDtypeStruct((N, 2 * C), jnp.float32),
        )(acc, h,
          lp['Wc1'], lp['bc1'].reshape(1, 2 * C),
          lp['gc'].reshape(1, 2 * C), lp['bc'].reshape(1, 2 * C),
          lp['Wc2'], lp['bc2'].reshape(1, C),
          lp['Wm1'], lp['bm1'].reshape(1, 2 * C),
          lp['gm'].reshape(1, 2 * C), lp['bm'].reshape(1, 2 * C),
          lp['Wm2'], lp['bm2'].reshape(1, C))

    out = pl.pallas_call(
        _pool_body,
        out_shape=jax.ShapeDtypeStruct((N_GRAPHS, 1), jnp.float32),
    )(h, batch.reshape(1, N), params['W_out'], params['b_out'].reshape(1, 1))
    return out


# speedup vs baseline: 8.5411x; 1.5675x over previous
import functools

import jax
import jax.numpy as jnp
from jax import lax
from jax.experimental import pallas as pl
from jax.experimental.pallas import tpu as pltpu
from jax.experimental.pallas import tpu_sc as plsc

N = 10000
E = 320000
X_DIM = 128
EDGE_DIM = 16
C = 64
N_GRAPHS = 128
EPS_GEN = 1e-7
BN_EPS = 1e-5

NC = 2
NS = 16
NW = NC * NS
K = 64
CPT = 158
EPAD = NW * K * CPT
DUMMY = N
ACC_ROWS = 10112
RPS = ACC_ROWS // NS


def _edge_pass_body(h_hbm, ea_hbm, src_hbm, dst_hbm, t_hbm, out_hbm,
                    acc, srcA, dstA, eaA, hsA, srcB, dstB, eaB, hsB,
                    outv, zv, tv, gsemA, gsemB, lsemA, lsemB):
    c = lax.axis_index("c")
    s = lax.axis_index("s")
    wid = c * NS + s

    pltpu.sync_copy(t_hbm, tv)

    zero = jnp.zeros((16,), jnp.float32)

    def _zb(i, carry):
        r = i // 8
        j = i - r * 8
        zv[r, pl.ds(j * 16, 16)] = zero
        return carry

    lax.fori_loop(0, 8 * 8, _zb, 0)
    row0 = s * RPS

    def _zc(i, carry):
        pltpu.sync_copy(zv, acc.at[pl.ds(row0 + i * 8, 8), :])
        return carry

    lax.fori_loop(0, RPS // 8, _zc, 0)
    plsc.subcore_barrier()

    tvec = tv[...]
    base0 = wid * CPT * K

    def _ld(t, sv, dv, ev, sem):
        b = base0 + t * K
        pltpu.async_copy(src_hbm.at[pl.ds(b, K)], sv, sem)
        pltpu.async_copy(dst_hbm.at[pl.ds(b, K)], dv, sem)
        pltpu.async_copy(ea_hbm.at[pl.ds(b, K), :], ev, sem)

    def _ldwait(t, sv, dv, ev, sem):
        b = base0 + t * K
        pltpu.make_async_copy(src_hbm.at[pl.ds(b, K)], sv, sem).wait()
        pltpu.make_async_copy(dst_hbm.at[pl.ds(b, K)], dv, sem).wait()
        pltpu.make_async_copy(ea_hbm.at[pl.ds(b, K), :], ev, sem).wait()

    def _compute(hsv, eav):
        @plsc.parallel_loop(0, K, unroll=4)
        def _cb(k):
            for j in range(4):
                hs = hsv[k, pl.ds(j * 16, 16)]
                ev = eav[k, pl.ds(j * 16, 16)]
                m = jnp.maximum(hs + ev, 0.0) + EPS_GEN
                e = jnp.exp(m * tvec)
                outv[k, pl.ds(j * 16, 16)] = m * e
                outv[k, pl.ds(C + j * 16, 16)] = e

    _ld(0, srcA, dstA, eaA, lsemA)
    _ldwait(0, srcA, dstA, eaA, lsemA)
    pltpu.async_copy(h_hbm.at[srcA], hsA, gsemA)
    _ld(1, srcB, dstB, eaB, lsemB)
    HALF = CPT // 2

    def _pipe(i, carry):
        t0 = 2 * i
        notlast = i < HALF - 1
        _ldwait(t0 + 1, srcB, dstB, eaB, lsemB)
        pltpu.async_copy(h_hbm.at[srcB], hsB, gsemB)
        pltpu.make_async_copy(h_hbm.at[srcA], hsA, gsemA).wait()
        _compute(hsA, eaA)
        pltpu.sync_copy(outv, acc.at[dstA], add=True)

        @pl.when(notlast)
        def _():
            _ld(t0 + 2, srcA, dstA, eaA, lsemA)
            _ldwait(t0 + 2, srcA, dstA, eaA, lsemA)
            pltpu.async_copy(h_hbm.at[srcA], hsA, gsemA)

        pltpu.make_async_copy(h_hbm.at[srcB], hsB, gsemB).wait()
        _compute(hsB, eaB)
        pltpu.sync_copy(outv, acc.at[dstB], add=True)

        @pl.when(notlast)
        def _():
            _ld(t0 + 3, srcB, dstB, eaB, lsemB)

        return carry

    lax.fori_loop(0, HALF, _pipe, 0)
    plsc.subcore_barrier()

    pltpu.sync_copy(acc.at[pl.ds(row0, RPS), :],
                    out_hbm.at[c, pl.ds(row0, RPS), :])


@functools.lru_cache(maxsize=1)
def _get_edge_pass():
  return pl.kernel(
    _edge_pass_body,
    out_type=jax.ShapeDtypeStruct((NC, ACC_ROWS, 2 * C), jnp.float32),
    mesh=plsc.VectorSubcoreMesh(
        core_axis_name="c", subcore_axis_name="s", num_cores=NC,
        num_subcores=NS),
    scratch_types=[
        pltpu.VMEM_SHARED((ACC_ROWS, 2 * C), jnp.float32),
        pltpu.VMEM((K,), jnp.int32),
        pltpu.VMEM((K,), jnp.int32),
        pltpu.VMEM((K, C), jnp.float32),
        pltpu.VMEM((K, 2 * C), jnp.float32),
        pltpu.VMEM((K,), jnp.int32),
        pltpu.VMEM((K,), jnp.int32),
        pltpu.VMEM((K, C), jnp.float32),
        pltpu.VMEM((K, 2 * C), jnp.float32),
        pltpu.VMEM((K, 2 * C), jnp.float32),
        pltpu.VMEM((8, 2 * C), jnp.float32),
        pltpu.VMEM((16,), jnp.float32),
        pltpu.SemaphoreType.DMA,
        pltpu.SemaphoreType.DMA,
        pltpu.SemaphoreType.DMA,
        pltpu.SemaphoreType.DMA,
    ],
  )


def _node_embed_body(x_ref, w_ref, b_ref, o_ref):
    o_ref[:, :C] = (
        jnp.dot(x_ref[...], w_ref[...], preferred_element_type=jnp.float32)
        + b_ref[...]
    )
    o_ref[:, C:] = jnp.zeros((N, C), jnp.float32)


def _edge_embed_body(a_ref, w_ref, b_ref, o_ref):
    o_ref[...] = (
        jnp.dot(a_ref[...], w_ref[...], preferred_element_type=jnp.float32)
        + b_ref[...]
    )


def _layer_body(acc_ref, h_ref, wc1, bc1, gc, bc, wc2, bc2,
                wm1, bm1, gm, bm, wm2, bm2, o_ref):
    numer = acc_ref[0, :N, :C] + acc_ref[1, :N, :C]
    denom = acc_ref[0, :N, C:] + acc_ref[1, :N, C:]
    h = h_ref[:, :C]
    aggr = numer / (denom + 1e-16)
    out = h + aggr
    hh = jnp.dot(out, wc1[...], preferred_element_type=jnp.float32) + bc1[...]
    mu = jnp.mean(hh, axis=0, keepdims=True)
    var = jnp.mean((hh - mu) ** 2, axis=0, keepdims=True)
    hh = (hh - mu) / jnp.sqrt(var + BN_EPS) * gc[...] + bc[...]
    hh = jnp.maximum(hh, 0.0)
    h2 = jnp.dot(hh, wc2[...], preferred_element_type=jnp.float32) + bc2[...]
    hm = jnp.dot(h2, wm1[...], preferred_element_type=jnp.float32) + bm1[...]
    mu2 = jnp.mean(hm, axis=0, keepdims=True)
    var2 = jnp.mean((hm - mu2) ** 2, axis=0, keepdims=True)
    hm = (hm - mu2) / jnp.sqrt(var2 + BN_EPS) * gm[...] + bm[...]
    hm = jnp.where(hm >= 0, hm, 0.01 * hm)
    h2 = jnp.dot(hm, wm2[...], preferred_element_type=jnp.float32) + bm2[...]
    o_ref[:, :C] = h2 + h
    o_ref[:, C:] = jnp.zeros((N, C), jnp.float32)


def _pool_body(h_ref, batch_ref, w_ref, b_ref, o_ref):
    gids = lax.broadcasted_iota(jnp.int32, (N_GRAPHS, 1), 0)
    bm = (batch_ref[...] == gids).astype(jnp.float32)
    ssum = jnp.dot(bm, h_ref[:, :C], preferred_element_type=jnp.float32)
    cnt = jnp.sum(bm, axis=1, keepdims=True)
    pooled = ssum / jnp.maximum(cnt, 1.0)
    o_ref[...] = (
        jnp.dot(pooled, w_ref[...], preferred_element_type=jnp.float32)
        + b_ref[...]
    )


def kernel(x, edge_index, edge_attr, batch, data, params):
    src = jnp.pad(edge_index[0], (0, EPAD - E))
    dst = jnp.pad(edge_index[1], (0, EPAD - E), constant_values=DUMMY)
    ea_in = jnp.pad(edge_attr, ((0, EPAD - E), (0, 0)))

    h = pl.pallas_call(
        _node_embed_body,
        out_shape=jax.ShapeDtypeStruct((N, 2 * C), jnp.float32),
    )(x, params['W_node'], params['b_node'].reshape(1, C))

    eb = 1536
    ea = pl.pallas_call(
        _edge_embed_body,
        grid=(EPAD // eb,),
        in_specs=[
            pl.BlockSpec((eb, EDGE_DIM), lambda i: (i, 0)),
            pl.BlockSpec((EDGE_DIM, C), lambda i: (0, 0)),
            pl.BlockSpec((1, C), lambda i: (0, 0)),
        ],
        out_specs=pl.BlockSpec((eb, C), lambda i: (i, 0)),
        out_shape=jax.ShapeDtypeStruct((EPAD, C), jnp.float32),
    )(ea_in, params['W_edge'], params['b_edge'].reshape(1, C))

    for lp in params['layers']:
        t_arr = jnp.broadcast_to(lp['t'], (16,)).astype(jnp.float32)
        acc = _get_edge_pass()(h, ea, src, dst, t_arr)
        h = pl.pallas_call(
            _layer_body,
            out_shape=jax.ShapeDtypeStruct((N, 2 * C), jnp.float32),
        )(acc, h,
          lp['Wc1'], lp['bc1'].reshape(1, 2 * C),
          lp['gc'].reshape(1, 2 * C), lp['bc'].reshape(1, 2 * C),
          lp['Wc2'], lp['bc2'].reshape(1, C),
          lp['Wm1'], lp['bm1'].reshape(1, 2 * C),
          lp['gm'].reshape(1, 2 * C), lp['bm'].reshape(1, 2 * C),
          lp['Wm2'], lp['bm2'].reshape(1, C))

    out = pl.pallas_call(
        _pool_body,
        out_shape=jax.ShapeDtypeStruct((N_GRAPHS, 1), jnp.float32),
    )(h, batch.reshape(1, N), params['W_out'], params['b_out'].reshape(1, 1))
    return out


# speedup vs baseline: 10.2788x; 1.2034x over previous
import functools

import jax
import jax.numpy as jnp
from jax import lax
from jax.experimental import pallas as pl
from jax.experimental.pallas import tpu as pltpu
from jax.experimental.pallas import tpu_sc as plsc

N = 10000
E = 320000
X_DIM = 128
EDGE_DIM = 16
C = 64
N_GRAPHS = 128
EPS_GEN = 1e-7
BN_EPS = 1e-5

NC = 2
NS = 16
NW = NC * NS
K = 64
SUP = 2
S = SUP * K
NSUP = 80
EPAD = NW * S * NSUP
TOTSUP = NW * NSUP
ACC_ROWS = 10112
RPS = ACC_ROWS // NS
NDUMMY = ACC_ROWS - N


def _edge_pass_body(h_hbm, ea_hbm, spk_hbm, t_hbm, out_hbm,
                    acc, sdA, sdB, eaA, eaB, hs0, hs1, outv, zv, tv,
                    gsem0, gsem1, lsemA, lsemB, zsem):
    c = lax.axis_index("c")
    s = lax.axis_index("s")
    wid = c * NS + s

    pltpu.sync_copy(t_hbm, tv)

    zero = jnp.zeros((16,), jnp.float32)

    def _zb(i, carry):
        r = i // 8
        j = i - r * 8
        zv[r, pl.ds(j * 16, 16)] = zero
        return carry

    lax.fori_loop(0, 8 * 8, _zb, 0)
    row0 = s * RPS

    def _zc(i, carry):
        pltpu.async_copy(zv, acc.at[pl.ds(row0 + i * 8, 8), :], zsem)
        return carry

    lax.fori_loop(0, RPS // 8, _zc, 0)

    def _zw(i, carry):
        pltpu.make_async_copy(zv, acc.at[pl.ds(row0 + i * 8, 8), :], zsem).wait()
        return carry

    lax.fori_loop(0, RPS // 8, _zw, 0)
    plsc.subcore_barrier()

    tvec = tv[...]
    sup0 = wid * NSUP

    def _ld(u, sd, ea, sem):
        pltpu.async_copy(spk_hbm.at[sup0 + u], sd, sem)
        pltpu.async_copy(ea_hbm.at[pl.ds((sup0 + u) * S, S), :], ea, sem)

    def _ldwait(u, sd, ea, sem):
        pltpu.make_async_copy(spk_hbm.at[sup0 + u], sd, sem).wait()
        pltpu.make_async_copy(
            ea_hbm.at[pl.ds((sup0 + u) * S, S), :], ea, sem).wait()

    def _compute(hsv, eav, g):
        @plsc.parallel_loop(0, K, unroll=4)
        def _cb(k):
            for j in range(4):
                hs = hsv[k, pl.ds(j * 16, 16)]
                ev = eav[g * K + k, pl.ds(j * 16, 16)]
                m = jnp.maximum(hs + ev, 0.0) + EPS_GEN
                e = jnp.exp(m * tvec)
                outv[k, pl.ds(j * 16, 16)] = m * e
                outv[k, pl.ds(C + j * 16, 16)] = e

    _ld(0, sdA, eaA, lsemA)
    _ldwait(0, sdA, eaA, lsemA)
    pltpu.async_copy(h_hbm.at[sdA.at[0]], hs0, gsem0)
    _ld(1, sdB, eaB, lsemB)
    PAIRS = NSUP // 2

    def _pipe(i, carry):
        u0 = 2 * i
        nl = i < PAIRS - 1

        pltpu.async_copy(h_hbm.at[sdA.at[1]], hs1, gsem1)
        pltpu.make_async_copy(h_hbm.at[sdA.at[0]], hs0, gsem0).wait()
        _compute(hs0, eaA, 0)
        pltpu.sync_copy(outv, acc.at[sdA.at[2]], add=True)
        _ldwait(u0 + 1, sdB, eaB, lsemB)
        pltpu.async_copy(h_hbm.at[sdB.at[0]], hs0, gsem0)
        pltpu.make_async_copy(h_hbm.at[sdA.at[1]], hs1, gsem1).wait()
        _compute(hs1, eaA, 1)
        pltpu.sync_copy(outv, acc.at[sdA.at[3]], add=True)

        @pl.when(nl)
        def _():
            _ld(u0 + 2, sdA, eaA, lsemA)

        pltpu.async_copy(h_hbm.at[sdB.at[1]], hs1, gsem1)
        pltpu.make_async_copy(h_hbm.at[sdB.at[0]], hs0, gsem0).wait()
        _compute(hs0, eaB, 0)
        pltpu.sync_copy(outv, acc.at[sdB.at[2]], add=True)

        @pl.when(nl)
        def _():
            _ldwait(u0 + 2, sdA, eaA, lsemA)
            pltpu.async_copy(h_hbm.at[sdA.at[0]], hs0, gsem0)

        pltpu.make_async_copy(h_hbm.at[sdB.at[1]], hs1, gsem1).wait()
        _compute(hs1, eaB, 1)
        pltpu.sync_copy(outv, acc.at[sdB.at[3]], add=True)

        @pl.when(nl)
        def _():
            _ld(u0 + 3, sdB, eaB, lsemB)

        return carry

    lax.fori_loop(0, PAIRS, _pipe, 0)
    plsc.subcore_barrier()

    pltpu.sync_copy(acc.at[pl.ds(row0, RPS), :],
                    out_hbm.at[c, pl.ds(row0, RPS), :])


@functools.lru_cache(maxsize=1)
def _get_edge_pass():
  return pl.kernel(
    _edge_pass_body,
    out_type=jax.ShapeDtypeStruct((NC, ACC_ROWS, 2 * C), jnp.float32),
    mesh=plsc.VectorSubcoreMesh(
        core_axis_name="c", subcore_axis_name="s", num_cores=NC,
        num_subcores=NS),
    compiler_params=pltpu.CompilerParams(use_tc_tiling_on_sc=False),
    scratch_types=[
        pltpu.VMEM_SHARED((ACC_ROWS, 2 * C), jnp.float32),
        pltpu.VMEM((4, K), jnp.int32),
        pltpu.VMEM((4, K), jnp.int32),
        pltpu.VMEM((S, C), jnp.float32),
        pltpu.VMEM((S, C), jnp.float32),
        pltpu.VMEM((K, C), jnp.float32),
        pltpu.VMEM((K, C), jnp.float32),
        pltpu.VMEM((K, 2 * C), jnp.float32),
        pltpu.VMEM((8, 2 * C), jnp.float32),
        pltpu.VMEM((16,), jnp.float32),
        pltpu.SemaphoreType.DMA,
        pltpu.SemaphoreType.DMA,
        pltpu.SemaphoreType.DMA,
        pltpu.SemaphoreType.DMA,
        pltpu.SemaphoreType.DMA,
    ],
  )


def _node_embed_body(x_ref, w_ref, b_ref, o_ref):
    o_ref[...] = (
        jnp.dot(x_ref[...], w_ref[...], preferred_element_type=jnp.float32)
        + b_ref[...]
    )


def _edge_embed_body(a_ref, w_ref, b_ref, o_ref):
    o_ref[...] = (
        jnp.dot(a_ref[...], w_ref[...], preferred_element_type=jnp.float32)
        + b_ref[...]
    )


def _layer_body(acc_ref, h_ref, wc1, bc1, gc, bc, wc2, bc2,
                wm1, bm1, gm, bm, wm2, bm2, o_ref):
    numer = acc_ref[0, :N, :C] + acc_ref[1, :N, :C]
    denom = acc_ref[0, :N, C:] + acc_ref[1, :N, C:]
    h = h_ref[...]
    aggr = numer / (denom + 1e-16)
    out = h + aggr
    hh = jnp.dot(out, wc1[...], preferred_element_type=jnp.float32) + bc1[...]
    mu = jnp.mean(hh, axis=0, keepdims=True)
    var = jnp.mean((hh - mu) ** 2, axis=0, keepdims=True)
    hh = (hh - mu) / jnp.sqrt(var + BN_EPS) * gc[...] + bc[...]
    hh = jnp.maximum(hh, 0.0)
    h2 = jnp.dot(hh, wc2[...], preferred_element_type=jnp.float32) + bc2[...]
    hm = jnp.dot(h2, wm1[...], preferred_element_type=jnp.float32) + bm1[...]
    mu2 = jnp.mean(hm, axis=0, keepdims=True)
    var2 = jnp.mean((hm - mu2) ** 2, axis=0, keepdims=True)
    hm = (hm - mu2) / jnp.sqrt(var2 + BN_EPS) * gm[...] + bm[...]
    hm = jnp.where(hm >= 0, hm, 0.01 * hm)
    h2 = jnp.dot(hm, wm2[...], preferred_element_type=jnp.float32) + bm2[...]
    o_ref[...] = h2 + h


def _pool_body(h_ref, batch_ref, w_ref, b_ref, o_ref):
    gids = lax.broadcasted_iota(jnp.int32, (N_GRAPHS, 1), 0)
    bm = (batch_ref[...] == gids).astype(jnp.float32)
    ssum = jnp.dot(bm, h_ref[...], preferred_element_type=jnp.float32)
    cnt = jnp.sum(bm, axis=1, keepdims=True)
    pooled = ssum / jnp.maximum(cnt, 1.0)
    o_ref[...] = (
        jnp.dot(pooled, w_ref[...], preferred_element_type=jnp.float32)
        + b_ref[...]
    )


def kernel(x, edge_index, edge_attr, batch, data, params):
    npad = EPAD - E
    src = jnp.pad(edge_index[0], (0, npad))
    dst = jnp.concatenate(
        [edge_index[1], N + (jnp.arange(npad, dtype=jnp.int32) % NDUMMY)])
    spk = jnp.concatenate(
        [src.reshape(TOTSUP, SUP, K), dst.reshape(TOTSUP, SUP, K)], axis=1)
    ea_in = jnp.pad(edge_attr, ((0, npad), (0, 0)))

    h = pl.pallas_call(
        _node_embed_body,
        out_shape=jax.ShapeDtypeStruct((N, C), jnp.float32),
    )(x, params['W_node'], params['b_node'].reshape(1, C))

    eb = 2560
    ea = pl.pallas_call(
        _edge_embed_body,
        grid=(EPAD // eb,),
        in_specs=[
            pl.BlockSpec((eb, EDGE_DIM), lambda i: (i, 0)),
            pl.BlockSpec((EDGE_DIM, C), lambda i: (0, 0)),
            pl.BlockSpec((1, C), lambda i: (0, 0)),
        ],
        out_specs=pl.BlockSpec((eb, C), lambda i: (i, 0)),
        out_shape=jax.ShapeDtypeStruct((EPAD, C), jnp.float32),
    )(ea_in, params['W_edge'], params['b_edge'].reshape(1, C))

    for lp in params['layers']:
        t_arr = jnp.broadcast_to(lp['t'], (16,)).astype(jnp.float32)
        acc = _get_edge_pass()(h, ea, spk, t_arr)
        h = pl.pallas_call(
            _layer_body,
            out_shape=jax.ShapeDtypeStruct((N, C), jnp.float32),
        )(acc, h,
          lp['Wc1'], lp['bc1'].reshape(1, 2 * C),
          lp['gc'].reshape(1, 2 * C), lp['bc'].reshape(1, 2 * C),
          lp['Wc2'], lp['bc2'].reshape(1, C),
          lp['Wm1'], lp['bm1'].reshape(1, 2 * C),
          lp['gm'].reshape(1, 2 * C), lp['bm'].reshape(1, 2 * C),
          lp['Wm2'], lp['bm2'].reshape(1, C))

    out = pl.pallas_call(
        _pool_body,
        out_shape=jax.ShapeDtypeStruct((N_GRAPHS, 1), jnp.float32),
    )(h, batch.reshape(1, N), params['W_out'], params['b_out'].reshape(1, 1))
    return out


# speedup vs baseline: 10.6535x; 1.0365x over previous
import functools

import jax
import jax.numpy as jnp
from jax import lax
from jax.experimental import pallas as pl
from jax.experimental.pallas import tpu as pltpu
from jax.experimental.pallas import tpu_sc as plsc

N = 10000
E = 320000
X_DIM = 128
EDGE_DIM = 16
C = 64
N_GRAPHS = 128
EPS_GEN = 1e-7
BN_EPS = 1e-5

NC = 2
NS = 16
NW = NC * NS
K = 64
SUP = 2
S = SUP * K
NSUP = 80
EPAD = NW * S * NSUP
TOTSUP = NW * NSUP
ACC_ROWS = 10112
RPS = ACC_ROWS // NS
NDUMMY = ACC_ROWS - N


def _edge_pass_body(h_hbm, ea_hbm, src_hbm, dst_hbm, t_hbm, out_hbm,
                    acc, srcA, srcB, dstA, dstB, eaA, eaB, hs0, hs1, outv,
                    zv, tv, gsem0, gsem1, lsemA, lsemB, zsem):
    c = lax.axis_index("c")
    s = lax.axis_index("s")
    wid = c * NS + s

    pltpu.sync_copy(t_hbm, tv)

    zero = jnp.zeros((16,), jnp.float32)

    def _zb(i, carry):
        r = i // 8
        j = i - r * 8
        zv[r, pl.ds(j * 16, 16)] = zero
        return carry

    lax.fori_loop(0, 8 * 8, _zb, 0)
    row0 = s * RPS

    def _zc(i, carry):
        pltpu.async_copy(zv, acc.at[pl.ds(row0 + i * 8, 8), :], zsem)
        return carry

    lax.fori_loop(0, RPS // 8, _zc, 0)

    def _zw(i, carry):
        pltpu.make_async_copy(zv, acc.at[pl.ds(row0 + i * 8, 8), :], zsem).wait()
        return carry

    lax.fori_loop(0, RPS // 8, _zw, 0)
    plsc.subcore_barrier()

    tvec = tv[...]
    sup0 = wid * NSUP
    emax = E - S

    def _ld(u, sv, dv, ev, sem):
        b = (sup0 + u) * S
        pltpu.async_copy(src_hbm.at[pl.ds(b, S)], sv, sem)
        pltpu.async_copy(dst_hbm.at[pl.ds(b, S)], dv, sem)
        be = jnp.minimum(b, emax)
        pltpu.async_copy(ea_hbm.at[pl.ds(be, S), :], ev, sem)

    def _ldwait(u, sv, dv, ev, sem):
        b = (sup0 + u) * S
        pltpu.make_async_copy(src_hbm.at[pl.ds(b, S)], sv, sem).wait()
        pltpu.make_async_copy(dst_hbm.at[pl.ds(b, S)], dv, sem).wait()
        be = jnp.minimum(b, emax)
        pltpu.make_async_copy(ea_hbm.at[pl.ds(be, S), :], ev, sem).wait()

    def _compute(hsv, eav, g):
        @plsc.parallel_loop(0, K, unroll=4)
        def _cb(k):
            for j in range(4):
                hs = hsv[k, pl.ds(j * 16, 16)]
                ev = eav[g * K + k, pl.ds(j * 16, 16)]
                m = jnp.maximum(hs + ev, 0.0) + EPS_GEN
                e = jnp.exp(m * tvec)
                outv[g * K + k, pl.ds(j * 16, 16)] = m * e
                outv[g * K + k, pl.ds(C + j * 16, 16)] = e

    _ld(0, srcA, dstA, eaA, lsemA)
    _ldwait(0, srcA, dstA, eaA, lsemA)
    pltpu.async_copy(h_hbm.at[srcA.at[pl.ds(0, K)]], hs0, gsem0)
    _ld(1, srcB, dstB, eaB, lsemB)
    PAIRS = NSUP // 2

    def _pipe(i, carry):
        u0 = 2 * i
        nl = i < PAIRS - 1

        pltpu.async_copy(h_hbm.at[srcA.at[pl.ds(K, K)]], hs1, gsem1)
        pltpu.make_async_copy(h_hbm.at[srcA.at[pl.ds(0, K)]], hs0, gsem0).wait()
        _compute(hs0, eaA, 0)
        _ldwait(u0 + 1, srcB, dstB, eaB, lsemB)
        pltpu.async_copy(h_hbm.at[srcB.at[pl.ds(0, K)]], hs0, gsem0)
        pltpu.make_async_copy(h_hbm.at[srcA.at[pl.ds(K, K)]], hs1, gsem1).wait()
        _compute(hs1, eaA, 1)
        pltpu.sync_copy(outv, acc.at[dstA], add=True)

        @pl.when(nl)
        def _():
            _ld(u0 + 2, srcA, dstA, eaA, lsemA)

        pltpu.async_copy(h_hbm.at[srcB.at[pl.ds(K, K)]], hs1, gsem1)
        pltpu.make_async_copy(h_hbm.at[srcB.at[pl.ds(0, K)]], hs0, gsem0).wait()
        _compute(hs0, eaB, 0)

        @pl.when(nl)
        def _():
            _ldwait(u0 + 2, srcA, dstA, eaA, lsemA)
            pltpu.async_copy(h_hbm.at[srcA.at[pl.ds(0, K)]], hs0, gsem0)

        pltpu.make_async_copy(h_hbm.at[srcB.at[pl.ds(K, K)]], hs1, gsem1).wait()
        _compute(hs1, eaB, 1)
        pltpu.sync_copy(outv, acc.at[dstB], add=True)

        @pl.when(nl)
        def _():
            _ld(u0 + 3, srcB, dstB, eaB, lsemB)

        return carry

    lax.fori_loop(0, PAIRS, _pipe, 0)
    plsc.subcore_barrier()

    pltpu.sync_copy(acc.at[pl.ds(row0, RPS), :],
                    out_hbm.at[c, pl.ds(row0, RPS), :])


@functools.lru_cache(maxsize=1)
def _get_edge_pass():
  return pl.kernel(
    _edge_pass_body,
    out_type=jax.ShapeDtypeStruct((NC, ACC_ROWS, 2 * C), jnp.float32),
    mesh=plsc.VectorSubcoreMesh(
        core_axis_name="c", subcore_axis_name="s", num_cores=NC,
        num_subcores=NS),
    compiler_params=pltpu.CompilerParams(use_tc_tiling_on_sc=False),
    scratch_types=[
        pltpu.VMEM_SHARED((ACC_ROWS, 2 * C), jnp.float32),
        pltpu.VMEM((S,), jnp.int32),
        pltpu.VMEM((S,), jnp.int32),
        pltpu.VMEM((S,), jnp.int32),
        pltpu.VMEM((S,), jnp.int32),
        pltpu.VMEM((S, C), jnp.float32),
        pltpu.VMEM((S, C), jnp.float32),
        pltpu.VMEM((K, C), jnp.float32),
        pltpu.VMEM((K, C), jnp.float32),
        pltpu.VMEM((S, 2 * C), jnp.float32),
        pltpu.VMEM((8, 2 * C), jnp.float32),
        pltpu.VMEM((16,), jnp.float32),
        pltpu.SemaphoreType.DMA,
        pltpu.SemaphoreType.DMA,
        pltpu.SemaphoreType.DMA,
        pltpu.SemaphoreType.DMA,
        pltpu.SemaphoreType.DMA,
    ],
  )


def _node_embed_body(x_ref, w_ref, b_ref, o_ref):
    o_ref[...] = (
        jnp.dot(x_ref[...], w_ref[...], preferred_element_type=jnp.float32)
        + b_ref[...]
    )


def _edge_embed_body(a_ref, w_ref, b_ref, o_ref):
    o_ref[...] = (
        jnp.dot(a_ref[...], w_ref[...], preferred_element_type=jnp.float32)
        + b_ref[...]
    )


def _layer_body(acc_ref, h_ref, wc1, bc1, gc, bc, wc2, bc2,
                wm1, bm1, gm, bm, wm2, bm2, o_ref):
    numer = acc_ref[0, :N, :C] + acc_ref[1, :N, :C]
    denom = acc_ref[0, :N, C:] + acc_ref[1, :N, C:]
    h = h_ref[...]
    aggr = numer / (denom + 1e-16)
    out = h + aggr
    hh = jnp.dot(out, wc1[...], preferred_element_type=jnp.float32) + bc1[...]
    mu = jnp.mean(hh, axis=0, keepdims=True)
    var = jnp.mean((hh - mu) ** 2, axis=0, keepdims=True)
    hh = (hh - mu) / jnp.sqrt(var + BN_EPS) * gc[...] + bc[...]
    hh = jnp.maximum(hh, 0.0)
    h2 = jnp.dot(hh, wc2[...], preferred_element_type=jnp.float32) + bc2[...]
    hm = jnp.dot(h2, wm1[...], preferred_element_type=jnp.float32) + bm1[...]
    mu2 = jnp.mean(hm, axis=0, keepdims=True)
    var2 = jnp.mean((hm - mu2) ** 2, axis=0, keepdims=True)
    hm = (hm - mu2) / jnp.sqrt(var2 + BN_EPS) * gm[...] + bm[...]
    hm = jnp.where(hm >= 0, hm, 0.01 * hm)
    h2 = jnp.dot(hm, wm2[...], preferred_element_type=jnp.float32) + bm2[...]
    o_ref[...] = h2 + h


def _pool_body(h_ref, batch_ref, w_ref, b_ref, o_ref):
    gids = lax.broadcasted_iota(jnp.int32, (N_GRAPHS, 1), 0)
    bm = (batch_ref[...] == gids).astype(jnp.float32)
    ssum = jnp.dot(bm, h_ref[...], preferred_element_type=jnp.float32)
    cnt = jnp.sum(bm, axis=1, keepdims=True)
    pooled = ssum / jnp.maximum(cnt, 1.0)
    o_ref[...] = (
        jnp.dot(pooled, w_ref[...], preferred_element_type=jnp.float32)
        + b_ref[...]
    )


def kernel(x, edge_index, edge_attr, batch, data, params):
    npad = EPAD - E
    src = jnp.pad(edge_index[0], (0, npad))
    dst = jnp.concatenate(
        [edge_index[1], N + (jnp.arange(npad, dtype=jnp.int32) % NDUMMY)])

    h = pl.pallas_call(
        _node_embed_body,
        out_shape=jax.ShapeDtypeStruct((N, C), jnp.float32),
    )(x, params['W_node'], params['b_node'].reshape(1, C))

    eb = E // 16
    ea = pl.pallas_call(
        _edge_embed_body,
        grid=(E // eb,),
        in_specs=[
            pl.BlockSpec((eb, EDGE_DIM), lambda i: (i, 0)),
            pl.BlockSpec((EDGE_DIM, C), lambda i: (0, 0)),
            pl.BlockSpec((1, C), lambda i: (0, 0)),
        ],
        out_specs=pl.BlockSpec((eb, C), lambda i: (i, 0)),
        out_shape=jax.ShapeDtypeStruct((E, C), jnp.float32),
    )(edge_attr, params['W_edge'], params['b_edge'].reshape(1, C))

    for lp in params['layers']:
        t_arr = jnp.broadcast_to(lp['t'], (16,)).astype(jnp.float32)
        acc = _get_edge_pass()(h, ea, src, dst, t_arr)
        h = pl.pallas_call(
            _layer_body,
            out_shape=jax.ShapeDtypeStruct((N, C), jnp.float32),
        )(acc, h,
          lp['Wc1'], lp['bc1'].reshape(1, 2 * C),
          lp['gc'].reshape(1, 2 * C), lp['bc'].reshape(1, 2 * C),
          lp['Wc2'], lp['bc2'].reshape(1, C),
          lp['Wm1'], lp['bm1'].reshape(1, 2 * C),
          lp['gm'].reshape(1, 2 * C), lp['bm'].reshape(1, 2 * C),
          lp['Wm2'], lp['bm2'].reshape(1, C))

    out = pl.pallas_call(
        _pool_body,
        out_shape=jax.ShapeDtypeStruct((N_GRAPHS, 1), jnp.float32),
    )(h, batch.reshape(1, N), params['W_out'], params['b_out'].reshape(1, 1))
    return out


# speedup vs baseline: 13.9311x; 1.3077x over previous
import functools

import jax
import jax.numpy as jnp
from jax import lax
from jax.experimental import pallas as pl
from jax.experimental.pallas import tpu as pltpu
from jax.experimental.pallas import tpu_sc as plsc

N = 10000
E = 320000
X_DIM = 128
EDGE_DIM = 16
C = 64
N_GRAPHS = 128
EPS_GEN = 1e-7
BN_EPS = 1e-5

NC = 2
NS = 16
NW = NC * NS
K = 64
SUP = 2
S = SUP * K
NSUP = 80
EPAD = NW * S * NSUP
TOTSUP = NW * NSUP
ACC_ROWS = 10112
RPS = ACC_ROWS // NS
NDUMMY = ACC_ROWS - N


def _edge_pass_body(h_hbm, ea_hbm, src_hbm, dst_hbm, t_hbm, out_hbm,
                    acc, srcA, srcB, dstA, dstB, eaA, eaB, hs0, hs1, outv,
                    zv, tv, gsem0, gsem1, lsemA, lsemB, zsem):
    c = lax.axis_index("c")
    s = lax.axis_index("s")
    wid = c * NS + s

    pltpu.sync_copy(t_hbm, tv)

    zero = jnp.zeros((16,), jnp.float32)

    def _zb(i, carry):
        r = i // 8
        j = i - r * 8
        zv[r, pl.ds(j * 16, 16)] = zero
        return carry

    lax.fori_loop(0, 8 * 8, _zb, 0)
    row0 = s * RPS

    def _zc(i, carry):
        pltpu.async_copy(zv, acc.at[pl.ds(row0 + i * 8, 8), :], zsem)
        return carry

    lax.fori_loop(0, RPS // 8, _zc, 0)

    def _zw(i, carry):
        pltpu.make_async_copy(zv, acc.at[pl.ds(row0 + i * 8, 8), :], zsem).wait()
        return carry

    lax.fori_loop(0, RPS // 8, _zw, 0)
    plsc.subcore_barrier()

    tvec = tv[...]
    sup0 = wid * NSUP
    emax = E - S

    def _ld(u, sv, dv, ev, sem):
        b = (sup0 + u) * S
        pltpu.async_copy(src_hbm.at[pl.ds(b, S)], sv, sem)
        pltpu.async_copy(dst_hbm.at[pl.ds(b, S)], dv, sem)
        be = jnp.minimum(b, emax)
        pltpu.async_copy(ea_hbm.at[pl.ds(be, S), pl.ds(0, C)], ev, sem)

    def _ldwait(u, sv, dv, ev, sem):
        b = (sup0 + u) * S
        pltpu.make_async_copy(src_hbm.at[pl.ds(b, S)], sv, sem).wait()
        pltpu.make_async_copy(dst_hbm.at[pl.ds(b, S)], dv, sem).wait()
        be = jnp.minimum(b, emax)
        pltpu.make_async_copy(
            ea_hbm.at[pl.ds(be, S), pl.ds(0, C)], ev, sem).wait()

    def _compute(hsv, eav, g):
        @plsc.parallel_loop(0, K, unroll=4)
        def _cb(k):
            for j in range(4):
                hs = hsv[k, pl.ds(j * 16, 16)]
                ev = eav[g * K + k, pl.ds(j * 16, 16)]
                m = jnp.maximum(hs + ev, 0.0) + EPS_GEN
                e = jnp.exp(m * tvec)
                outv[g * K + k, pl.ds(j * 16, 16)] = m * e
                outv[g * K + k, pl.ds(C + j * 16, 16)] = e

    _ld(0, srcA, dstA, eaA, lsemA)
    _ldwait(0, srcA, dstA, eaA, lsemA)
    pltpu.async_copy(h_hbm.at[srcA.at[pl.ds(0, K)]], hs0, gsem0)
    _ld(1, srcB, dstB, eaB, lsemB)
    PAIRS = NSUP // 2

    def _pipe(i, carry):
        u0 = 2 * i
        nl = i < PAIRS - 1

        pltpu.async_copy(h_hbm.at[srcA.at[pl.ds(K, K)]], hs1, gsem1)
        pltpu.make_async_copy(h_hbm.at[srcA.at[pl.ds(0, K)]], hs0, gsem0).wait()
        _compute(hs0, eaA, 0)
        _ldwait(u0 + 1, srcB, dstB, eaB, lsemB)
        pltpu.async_copy(h_hbm.at[srcB.at[pl.ds(0, K)]], hs0, gsem0)
        pltpu.make_async_copy(h_hbm.at[srcA.at[pl.ds(K, K)]], hs1, gsem1).wait()
        _compute(hs1, eaA, 1)
        pltpu.sync_copy(outv, acc.at[dstA], add=True)

        @pl.when(nl)
        def _():
            _ld(u0 + 2, srcA, dstA, eaA, lsemA)

        pltpu.async_copy(h_hbm.at[srcB.at[pl.ds(K, K)]], hs1, gsem1)
        pltpu.make_async_copy(h_hbm.at[srcB.at[pl.ds(0, K)]], hs0, gsem0).wait()
        _compute(hs0, eaB, 0)

        @pl.when(nl)
        def _():
            _ldwait(u0 + 2, srcA, dstA, eaA, lsemA)
            pltpu.async_copy(h_hbm.at[srcA.at[pl.ds(0, K)]], hs0, gsem0)

        pltpu.make_async_copy(h_hbm.at[srcB.at[pl.ds(K, K)]], hs1, gsem1).wait()
        _compute(hs1, eaB, 1)
        pltpu.sync_copy(outv, acc.at[dstB], add=True)

        @pl.when(nl)
        def _():
            _ld(u0 + 3, srcB, dstB, eaB, lsemB)

        return carry

    lax.fori_loop(0, PAIRS, _pipe, 0)
    plsc.subcore_barrier()

    pltpu.sync_copy(acc.at[pl.ds(row0, RPS), :],
                    out_hbm.at[c, pl.ds(row0, RPS), :])


@functools.lru_cache(maxsize=1)
def _get_edge_pass():
  return pl.kernel(
    _edge_pass_body,
    out_type=jax.ShapeDtypeStruct((NC, ACC_ROWS, 2 * C), jnp.float32),
    mesh=plsc.VectorSubcoreMesh(
        core_axis_name="c", subcore_axis_name="s", num_cores=NC,
        num_subcores=NS),
    compiler_params=pltpu.CompilerParams(use_tc_tiling_on_sc=False),
    scratch_types=[
        pltpu.VMEM_SHARED((ACC_ROWS, 2 * C), jnp.float32),
        pltpu.VMEM((S,), jnp.int32),
        pltpu.VMEM((S,), jnp.int32),
        pltpu.VMEM((S,), jnp.int32),
        pltpu.VMEM((S,), jnp.int32),
        pltpu.VMEM((S, C), jnp.float32),
        pltpu.VMEM((S, C), jnp.float32),
        pltpu.VMEM((K, C), jnp.float32),
        pltpu.VMEM((K, C), jnp.float32),
        pltpu.VMEM((S, 2 * C), jnp.float32),
        pltpu.VMEM((8, 2 * C), jnp.float32),
        pltpu.VMEM((16,), jnp.float32),
        pltpu.SemaphoreType.DMA,
        pltpu.SemaphoreType.DMA,
        pltpu.SemaphoreType.DMA,
        pltpu.SemaphoreType.DMA,
        pltpu.SemaphoreType.DMA,
    ],
  )


def _node_embed_body(x_ref, w_ref, b_ref, o_ref):
    o_ref[...] = (
        jnp.dot(x_ref[...], w_ref[...], preferred_element_type=jnp.float32)
        + b_ref[...]
    )


def _edge_embed_body(at_ref, w_ref, b_ref, o_ref):
    y = lax.dot_general(
        at_ref[...], w_ref[...], (((0,), (0,)), ((), ())),
        preferred_element_type=jnp.float32) + b_ref[...]
    o_ref[:, :C] = y
    o_ref[:, C:] = jnp.zeros(y.shape, jnp.float32)


def _layer_body(acc_ref, h_ref, wc1, bc1, gc, bc, wc2, bc2,
                wm1, bm1, gm, bm, wm2, bm2, o_ref):
    numer = acc_ref[0, :N, :C] + acc_ref[1, :N, :C]
    denom = acc_ref[0, :N, C:] + acc_ref[1, :N, C:]
    h = h_ref[...]
    aggr = numer / (denom + 1e-16)
    out = h + aggr
    hh = jnp.dot(out, wc1[...], preferred_element_type=jnp.float32) + bc1[...]
    mu = jnp.mean(hh, axis=0, keepdims=True)
    var = jnp.mean((hh - mu) ** 2, axis=0, keepdims=True)
    hh = (hh - mu) / jnp.sqrt(var + BN_EPS) * gc[...] + bc[...]
    hh = jnp.maximum(hh, 0.0)
    h2 = jnp.dot(hh, wc2[...], preferred_element_type=jnp.float32) + bc2[...]
    hm = jnp.dot(h2, wm1[...], preferred_element_type=jnp.float32) + bm1[...]
    mu2 = jnp.mean(hm, axis=0, keepdims=True)
    var2 = jnp.mean((hm - mu2) ** 2, axis=0, keepdims=True)
    hm = (hm - mu2) / jnp.sqrt(var2 + BN_EPS) * gm[...] + bm[...]
    hm = jnp.where(hm >= 0, hm, 0.01 * hm)
    h2 = jnp.dot(hm, wm2[...], preferred_element_type=jnp.float32) + bm2[...]
    o_ref[...] = h2 + h


def _pool_body(h_ref, batch_ref, w_ref, b_ref, o_ref):
    gids = lax.broadcasted_iota(jnp.int32, (N_GRAPHS, 1), 0)
    bm = (batch_ref[...] == gids).astype(jnp.float32)
    ssum = jnp.dot(bm, h_ref[...], preferred_element_type=jnp.float32)
    cnt = jnp.sum(bm, axis=1, keepdims=True)
    pooled = ssum / jnp.maximum(cnt, 1.0)
    o_ref[...] = (
        jnp.dot(pooled, w_ref[...], preferred_element_type=jnp.float32)
        + b_ref[...]
    )


def kernel(x, edge_index, edge_attr, batch, data, params):
    npad = EPAD - E
    src = jnp.pad(edge_index[0], (0, npad))
    dst = jnp.concatenate(
        [edge_index[1], N + (jnp.arange(npad, dtype=jnp.int32) % NDUMMY)])

    h = pl.pallas_call(
        _node_embed_body,
        out_shape=jax.ShapeDtypeStruct((N, C), jnp.float32),
    )(x, params['W_node'], params['b_node'].reshape(1, C))

    eb = E // 10
    ea = pl.pallas_call(
        _edge_embed_body,
        grid=(E // eb,),
        in_specs=[
            pl.BlockSpec((EDGE_DIM, eb), lambda i: (0, i)),
            pl.BlockSpec((EDGE_DIM, C), lambda i: (0, 0)),
            pl.BlockSpec((1, C), lambda i: (0, 0)),
        ],
        out_specs=pl.BlockSpec((eb, 2 * C), lambda i: (i, 0)),
        out_shape=jax.ShapeDtypeStruct((E, 2 * C), jnp.float32),
    )(edge_attr.T, params['W_edge'], params['b_edge'].reshape(1, C))

    for lp in params['layers']:
        t_arr = jnp.broadcast_to(lp['t'], (16,)).astype(jnp.float32)
        acc = _get_edge_pass()(h, ea, src, dst, t_arr)
        h = pl.pallas_call(
            _layer_body,
            out_shape=jax.ShapeDtypeStruct((N, C), jnp.float32),
        )(acc, h,
          lp['Wc1'], lp['bc1'].reshape(1, 2 * C),
          lp['gc'].reshape(1, 2 * C), lp['bc'].reshape(1, 2 * C),
          lp['Wc2'], lp['bc2'].reshape(1, C),
          lp['Wm1'], lp['bm1'].reshape(1, 2 * C),
          lp['gm'].reshape(1, 2 * C), lp['bm'].reshape(1, 2 * C),
          lp['Wm2'], lp['bm2'].reshape(1, C))

    out = pl.pallas_call(
        _pool_body,
        out_shape=jax.ShapeDtypeStruct((N_GRAPHS, 1), jnp.float32),
    )(h, batch.reshape(1, N), params['W_out'], params['b_out'].reshape(1, 1))
    return out
